# baseline (speedup 1.0000x reference)
import functools

import jax
import jax.numpy as jnp
from jax import lax
from jax.experimental import pallas as pl
from jax.experimental.pallas import tpu as pltpu
from jax.experimental.pallas import tpu_sc as plsc

N = 50000
DIN = 128
H = 64
HH = H // 2
NSUB = 16
NACC = 50048
ROWS_PER_TILE = NACC // NSUB
ZCH = 136
NZ = ROWS_PER_TILE // ZCH
C = 128
RBLK = 2000


def _spmm_body(ep, src, zeros, out, e_v, idx_v, gath_v, scat_v, acc,
               sem_st, sem_g, sem_sc):
    c = lax.axis_index("c")
    s = lax.axis_index("s")
    M = src.shape[0] // 2
    off = c * M
    CPT = ep.shape[0] // NSUB
    base = s * CPT

    row0 = pl.multiple_of(s * ROWS_PER_TILE, 8)
    pltpu.sync_copy(zeros, acc.at[pl.ds(row0, ROWS_PER_TILE)])
    plsc.subcore_barrier()

    def stage(k, eb):
        pltpu.async_copy(ep.at[base + k], e_v.at[eb], sem_st)

    def wait_stage():
        pltpu.make_async_copy(ep.at[base], e_v.at[0], sem_st).wait()

    def issue_gather(b, eb):
        for i in range(C // 16):
            sl = pl.ds(i * 16, 16)
            idx_v[b, sl] = e_v[eb, 1, sl] + off
        pltpu.async_copy(src.at[idx_v.at[b]], gath_v.at[b], sem_g)

    def wait_gather():
        pltpu.make_async_copy(src.at[idx_v.at[0]], gath_v.at[0],
                              sem_g).wait()

    def issue_scatter(b, eb):
        pltpu.async_copy(scat_v.at[b], acc.at[e_v.at[eb, 0]], sem_sc,
                         add=True)

    def wait_scatter():
        pltpu.make_async_copy(scat_v.at[0], acc.at[e_v.at[0, 0]],
                              sem_sc).wait()

    stage(0, 0)
    stage(1, 1)
    wait_stage()
    issue_gather(0, 0)

    def iter_body(k, carry):
        b = lax.rem(k, 2)
        eb = lax.rem(k, 3)

        @pl.when(k + 1 < CPT)
        def _():
            wait_stage()

            @pl.when(k >= 1)
            def _():
                wait_scatter()

            issue_gather(lax.rem(k + 1, 2), lax.rem(k + 1, 3))

        @pl.when(k + 2 < CPT)
        def _():
            stage(k + 2, lax.rem(k + 2, 3))

        wait_gather()

        def cgrp(g, carry2):
            for h in range(2):
                e0 = g * 32 + h * 16
                vv = plsc.bitcast(e_v[eb, 2, pl.ds(e0, 16)], jnp.float32)
                for t in range(16):
                    v = vv[t]
                    r = e0 + t
                    for j in range(HH // 16):
                        sl = pl.ds(j * 16, 16)
                        scat_v[b, r, sl] = gath_v[b, r, sl] * v
            return carry2

        lax.fori_loop(0, C // 32, cgrp, 0)
        issue_scatter(b, eb)
        return carry

    lax.fori_loop(0, CPT, iter_body, 0)
    wait_scatter()
    wait_scatter()
    plsc.subcore_barrier()
    LAST = N - 15 * ROWS_PER_TILE

    @pl.when(s < NSUB - 1)
    def _():
        pltpu.sync_copy(acc.at[pl.ds(row0, ROWS_PER_TILE)],
                        out.at[pl.ds(c * N + row0, ROWS_PER_TILE)])

    @pl.when(s == NSUB - 1)
    def _():
        pltpu.sync_copy(acc.at[pl.ds(row0, LAST)],
                        out.at[pl.ds(c * N + row0, LAST)])


def _pack_edges(rows, cols, vals):
    E = rows.shape[0]
    Ep = -(-E // (NSUB * C)) * (NSUB * C)
    pad = Ep - E
    if pad:
        rows = jnp.concatenate([rows, jnp.zeros((pad,), jnp.int32)])
        cols = jnp.concatenate([cols, jnp.zeros((pad,), jnp.int32)])
        vals = jnp.concatenate([vals, jnp.zeros((pad,), jnp.float32)])
    vb = lax.bitcast_convert_type(vals, jnp.int32)
    return jnp.stack([rows.reshape(-1, C), cols.reshape(-1, C),
                      vb.reshape(-1, C)], axis=1)


def _spmm_sc(rows, cols, vals, src, dep=None):
    ep = _pack_edges(rows, cols, vals)
    mesh = plsc.VectorSubcoreMesh(core_axis_name="c", subcore_axis_name="s")
    kern = functools.partial(
        pl.kernel,
        out_type=jax.ShapeDtypeStruct((2 * N, HH), jnp.float32),
        mesh=mesh,
        compiler_params=pltpu.CompilerParams(use_tc_tiling_on_sc=False,
                                             needs_layout_passes=False),
        scratch_types=[
            pltpu.VMEM((3, 3, C), jnp.int32),
            pltpu.VMEM((2, C), jnp.int32),
            pltpu.VMEM((2, C, HH), jnp.float32),
            pltpu.VMEM((2, C, HH), jnp.float32),
            pltpu.VMEM_SHARED((NACC, HH), jnp.float32),
            pltpu.SemaphoreType.DMA,
            pltpu.SemaphoreType.DMA,
            pltpu.SemaphoreType.DMA,
        ],
    )(_spmm_body)
    zeros = jnp.zeros((ROWS_PER_TILE, HH), jnp.float32)
    if dep is not None:
        zeros, _ = lax.optimization_barrier((zeros, dep[0, :1]))
    return kern(ep, src, zeros)


def _prologue_body(nf, nt, W0, b0, Wa0, ba0, Wa1, ba1, x0_out, x1_out):
    hid = jnp.dot(nf[...], W0[...], preferred_element_type=jnp.float32)
    hid = hid + b0[...]
    hid = jnp.where(nt[...] == 0, hid, 0.0)
    x0 = jnp.dot(hid, Wa0[...], preferred_element_type=jnp.float32) + ba0[...]
    x1 = jnp.dot(hid, Wa1[...], preferred_element_type=jnp.float32) + ba1[...]
    x0_out[0] = x0[:, :HH]
    x0_out[1] = x0[:, HH:]
    x1_out[0] = x1[:, :HH]
    x1_out[1] = x1[:, HH:]


def _prologue(nf, nt, W0, b0, Wa0, ba0, Wa1, ba1):
    grid = (N // RBLK,)
    full = lambda shape: pl.BlockSpec(shape, lambda i: (0,) * len(shape))
    out_spec = pl.BlockSpec((2, RBLK, HH), lambda i: (0, i, 0))
    return pl.pallas_call(
        _prologue_body,
        grid=grid,
        in_specs=[
            pl.BlockSpec((RBLK, DIN), lambda i: (i, 0)),
            pl.BlockSpec((RBLK, 1), lambda i: (i, 0)),
            full((DIN, H)),
            full((1, H)),
            full((H, H)),
            full((1, H)),
            full((H, H)),
            full((1, H)),
        ],
        out_specs=[out_spec, out_spec],
        out_shape=[
            jax.ShapeDtypeStruct((2, N, HH), jnp.float32),
            jax.ShapeDtypeStruct((2, N, HH), jnp.float32),
        ],
    )(nf, nt, W0, b0, Wa0, ba0, Wa1, ba1)


def _layernorm(h, g, b):
    mu = jnp.mean(h, axis=-1, keepdims=True)
    var = jnp.mean((h - mu) ** 2, axis=-1, keepdims=True)
    return (h - mu) * lax.rsqrt(var + 1e-5) * g + b


def _epilogue_body(s20, s21, g0, bb0, g1, bb1, W1, b1, W2, b2, out):
    h0 = jnp.concatenate([s20[0], s20[1]], axis=-1)
    h1 = jnp.concatenate([s21[0], s21[1]], axis=-1)
    h0 = _layernorm(h0, g0[...], bb0[...])
    h1 = _layernorm(h1, g1[...], bb1[...])
    t0 = jnp.tanh(jnp.dot(h0, W1[...], preferred_element_type=jnp.float32)
                  + b1[...])
    t1 = jnp.tanh(jnp.dot(h1, W1[...], preferred_element_type=jnp.float32)
                  + b1[...])
    a0 = jnp.dot(t0, W2[...], preferred_element_type=jnp.float32) + b2[...]
    a1 = jnp.dot(t1, W2[...], preferred_element_type=jnp.float32) + b2[...]
    mx = jnp.maximum(a0, a1)
    e0 = jnp.exp(a0 - mx)
    e1 = jnp.exp(a1 - mx)
    z = e0 + e1
    out[...] = (e0 * h0 + e1 * h1) / z


def _epilogue(s20, s21, g0, bb0, g1, bb1, W1, b1, W2, b2):
    grid = (N // RBLK,)
    full = lambda shape: pl.BlockSpec(shape, lambda i: (0,) * len(shape))
    in_spec = pl.BlockSpec((2, RBLK, HH), lambda i: (0, i, 0))
    return pl.pallas_call(
        _epilogue_body,
        grid=grid,
        in_specs=[
            in_spec,
            in_spec,
            full((1, H)),
            full((1, H)),
            full((1, H)),
            full((1, H)),
            full((H, H)),
            full((1, H)),
            full((H, 1)),
            full((1, 1)),
        ],
        out_specs=pl.BlockSpec((RBLK, H), lambda i: (i, 0)),
        out_shape=jax.ShapeDtypeStruct((N, H), jnp.float32),
    )(s20, s21, g0, bb0, g1, bb1, W1, b1, W2, b2)


def kernel(node_feats, node_types, adj_indices, adj_values, idxes_seq,
           idxes_res, cosins, semantics, W0, b0, Wa_0, ba_0, gamma_0, beta_0,
           Wa_1, ba_1, gamma_1, beta_1, attn_W1, attn_b1, attn_W2, attn_b2):
    nt = node_types.reshape(N, 1)
    x0, x1 = _prologue(node_feats, nt, W0, b0.reshape(1, H),
                       Wa_0, ba_0.reshape(1, H), Wa_1, ba_1.reshape(1, H))
    xs = (x0, x1)
    s2s = []
    dep = None
    for m in range(2):
        xm = xs[m]
        xf = xm.reshape(2 * N, HH)
        k0 = idxes_seq[m, 0]
        k1 = idxes_seq[m, 1]
        r = idxes_res[m, 0]
        s1 = _spmm_sc(adj_indices[k0, 0], adj_indices[k0, 1],
                      adj_values[k0], xf, dep=dep)
        tbl = jnp.concatenate([s1.reshape(2, N, HH), xm],
                              axis=1).reshape(4 * N, HH)
        rows_c = jnp.concatenate([adj_indices[k1, 0], adj_indices[r, 0]])
        cols_c = jnp.concatenate([adj_indices[k1, 1], adj_indices[r, 1] + N])
        vals_c = jnp.concatenate([adj_values[k1], adj_values[r]])
        s2 = _spmm_sc(rows_c, cols_c, vals_c, tbl, dep=s1)
        s2s.append(s2.reshape(2, N, HH))
        dep = s2
    return _epilogue(s2s[0], s2s[1], gamma_0.reshape(1, H),
                     beta_0.reshape(1, H), gamma_1.reshape(1, H),
                     beta_1.reshape(1, H), attn_W1, attn_b1.reshape(1, H),
                     attn_W2, attn_b2.reshape(1, 1))


# speedup vs baseline: 1.0170x; 1.0170x over previous
import functools

import jax
import jax.numpy as jnp
from jax import lax
from jax.experimental import pallas as pl
from jax.experimental.pallas import tpu as pltpu
from jax.experimental.pallas import tpu_sc as plsc

N = 50000
DIN = 128
H = 64
HH = H // 2
NSUB = 16
NACC = 50048
ROWS_PER_TILE = NACC // NSUB
ZCH = 136
NZ = ROWS_PER_TILE // ZCH
C = 128
RBLK = 2000


def _spmm_body(ep, src, zeros, out, e_v, gath_v, scat_v, acc,
               sem_st, sem_g, sem_sc):
    c = lax.axis_index("c")
    s = lax.axis_index("s")
    M = src.shape[0] // 2
    off = c * M
    CPT = ep.shape[0] // NSUB
    base = s * CPT

    row0 = pl.multiple_of(s * ROWS_PER_TILE, 8)
    pltpu.sync_copy(zeros, acc.at[pl.ds(row0, ROWS_PER_TILE)])
    plsc.subcore_barrier()

    def stage(k, eb):
        pltpu.async_copy(ep.at[base + k], e_v.at[eb], sem_st)

    def wait_stage():
        pltpu.make_async_copy(ep.at[base], e_v.at[0], sem_st).wait()

    def issue_gather(b, eb):
        @pl.when(c == 0)
        def _():
            pltpu.async_copy(src.at[e_v.at[eb, 1]], gath_v.at[b], sem_g)

        @pl.when(c == 1)
        def _():
            pltpu.async_copy(src.at[e_v.at[eb, 2]], gath_v.at[b], sem_g)

    def wait_gather():
        pltpu.make_async_copy(src.at[e_v.at[0, 1]], gath_v.at[0],
                              sem_g).wait()

    def issue_scatter(b, eb):
        pltpu.async_copy(scat_v.at[b], acc.at[e_v.at[eb, 0]], sem_sc,
                         add=True)

    def wait_scatter():
        pltpu.make_async_copy(scat_v.at[0], acc.at[e_v.at[0, 0]],
                              sem_sc).wait()

    stage(0, 0)
    stage(1, 1)
    wait_stage()
    issue_gather(0, 0)

    def iter_body(k, carry):
        b = lax.rem(k, 2)
        eb = lax.rem(k, 3)

        @pl.when(k + 1 < CPT)
        def _():
            wait_stage()

            @pl.when(k >= 1)
            def _():
                wait_scatter()

            issue_gather(lax.rem(k + 1, 2), lax.rem(k + 1, 3))

        @pl.when(k + 2 < CPT)
        def _():
            stage(k + 2, lax.rem(k + 2, 3))

        wait_gather()

        def cgrp(g, carry2):
            for h in range(2):
                e0 = g * 32 + h * 16
                vv = plsc.bitcast(e_v[eb, 3, pl.ds(e0, 16)], jnp.float32)
                for t in range(16):
                    v = vv[t]
                    r = e0 + t
                    for j in range(HH // 16):
                        sl = pl.ds(j * 16, 16)
                        scat_v[b, r, sl] = gath_v[b, r, sl] * v
            return carry2

        lax.fori_loop(0, C // 32, cgrp, 0)
        issue_scatter(b, eb)
        return carry

    lax.fori_loop(0, CPT, iter_body, 0)
    wait_scatter()
    wait_scatter()
    plsc.subcore_barrier()
    LAST = N - 15 * ROWS_PER_TILE

    @pl.when(s < NSUB - 1)
    def _():
        pltpu.sync_copy(acc.at[pl.ds(row0, ROWS_PER_TILE)],
                        out.at[pl.ds(c * N + row0, ROWS_PER_TILE)])

    @pl.when(s == NSUB - 1)
    def _():
        pltpu.sync_copy(acc.at[pl.ds(row0, LAST)],
                        out.at[pl.ds(c * N + row0, LAST)])


def _pack_edges(rows, cols, vals, M):
    E = rows.shape[0]
    Ep = -(-E // (NSUB * C)) * (NSUB * C)
    pad = Ep - E
    if pad:
        rows = jnp.concatenate([rows, jnp.zeros((pad,), jnp.int32)])
        cols = jnp.concatenate([cols, jnp.zeros((pad,), jnp.int32)])
        vals = jnp.concatenate([vals, jnp.zeros((pad,), jnp.float32)])
    vb = lax.bitcast_convert_type(vals, jnp.int32)
    return jnp.stack([rows.reshape(-1, C), cols.reshape(-1, C),
                      (cols + M).reshape(-1, C), vb.reshape(-1, C)], axis=1)


def _spmm_sc(rows, cols, vals, src, dep=None):
    M = src.shape[0] // 2
    ep = _pack_edges(rows, cols, vals, M)
    mesh = plsc.VectorSubcoreMesh(core_axis_name="c", subcore_axis_name="s")
    kern = functools.partial(
        pl.kernel,
        out_type=jax.ShapeDtypeStruct((2 * N, HH), jnp.float32),
        mesh=mesh,
        compiler_params=pltpu.CompilerParams(use_tc_tiling_on_sc=False,
                                             needs_layout_passes=False),
        scratch_types=[
            pltpu.VMEM((3, 4, C), jnp.int32),
            pltpu.VMEM((2, C, HH), jnp.float32),
            pltpu.VMEM((2, C, HH), jnp.float32),
            pltpu.VMEM_SHARED((NACC, HH), jnp.float32),
            pltpu.SemaphoreType.DMA,
            pltpu.SemaphoreType.DMA,
            pltpu.SemaphoreType.DMA,
        ],
    )(_spmm_body)
    zeros = jnp.zeros((ROWS_PER_TILE, HH), jnp.float32)
    if dep is not None:
        zeros, _ = lax.optimization_barrier((zeros, dep[0, :1]))
    return kern(ep, src, zeros)


def _prologue_body(nf, nt, W0, b0, Wa0, ba0, Wa1, ba1, x0_out, x1_out):
    hid = jnp.dot(nf[...], W0[...], preferred_element_type=jnp.float32)
    hid = hid + b0[...]
    hid = jnp.where(nt[...] == 0, hid, 0.0)
    x0 = jnp.dot(hid, Wa0[...], preferred_element_type=jnp.float32) + ba0[...]
    x1 = jnp.dot(hid, Wa1[...], preferred_element_type=jnp.float32) + ba1[...]
    x0_out[0] = x0[:, :HH]
    x0_out[1] = x0[:, HH:]
    x1_out[0] = x1[:, :HH]
    x1_out[1] = x1[:, HH:]


def _prologue(nf, nt, W0, b0, Wa0, ba0, Wa1, ba1):
    grid = (N // RBLK,)
    full = lambda shape: pl.BlockSpec(shape, lambda i: (0,) * len(shape))
    out_spec = pl.BlockSpec((2, RBLK, HH), lambda i: (0, i, 0))
    return pl.pallas_call(
        _prologue_body,
        grid=grid,
        in_specs=[
            pl.BlockSpec((RBLK, DIN), lambda i: (i, 0)),
            pl.BlockSpec((RBLK, 1), lambda i: (i, 0)),
            full((DIN, H)),
            full((1, H)),
            full((H, H)),
            full((1, H)),
            full((H, H)),
            full((1, H)),
        ],
        out_specs=[out_spec, out_spec],
        out_shape=[
            jax.ShapeDtypeStruct((2, N, HH), jnp.float32),
            jax.ShapeDtypeStruct((2, N, HH), jnp.float32),
        ],
    )(nf, nt, W0, b0, Wa0, ba0, Wa1, ba1)


def _layernorm(h, g, b):
    mu = jnp.mean(h, axis=-1, keepdims=True)
    var = jnp.mean((h - mu) ** 2, axis=-1, keepdims=True)
    return (h - mu) * lax.rsqrt(var + 1e-5) * g + b


def _epilogue_body(s20, s21, g0, bb0, g1, bb1, W1, b1, W2, b2, out):
    h0 = jnp.concatenate([s20[0], s20[1]], axis=-1)
    h1 = jnp.concatenate([s21[0], s21[1]], axis=-1)
    h0 = _layernorm(h0, g0[...], bb0[...])
    h1 = _layernorm(h1, g1[...], bb1[...])
    t0 = jnp.tanh(jnp.dot(h0, W1[...], preferred_element_type=jnp.float32)
                  + b1[...])
    t1 = jnp.tanh(jnp.dot(h1, W1[...], preferred_element_type=jnp.float32)
                  + b1[...])
    a0 = jnp.dot(t0, W2[...], preferred_element_type=jnp.float32) + b2[...]
    a1 = jnp.dot(t1, W2[...], preferred_element_type=jnp.float32) + b2[...]
    mx = jnp.maximum(a0, a1)
    e0 = jnp.exp(a0 - mx)
    e1 = jnp.exp(a1 - mx)
    z = e0 + e1
    out[...] = (e0 * h0 + e1 * h1) / z


def _epilogue(s20, s21, g0, bb0, g1, bb1, W1, b1, W2, b2):
    grid = (N // RBLK,)
    full = lambda shape: pl.BlockSpec(shape, lambda i: (0,) * len(shape))
    in_spec = pl.BlockSpec((2, RBLK, HH), lambda i: (0, i, 0))
    return pl.pallas_call(
        _epilogue_body,
        grid=grid,
        in_specs=[
            in_spec,
            in_spec,
            full((1, H)),
            full((1, H)),
            full((1, H)),
            full((1, H)),
            full((H, H)),
            full((1, H)),
            full((H, 1)),
            full((1, 1)),
        ],
        out_specs=pl.BlockSpec((RBLK, H), lambda i: (i, 0)),
        out_shape=jax.ShapeDtypeStruct((N, H), jnp.float32),
    )(s20, s21, g0, bb0, g1, bb1, W1, b1, W2, b2)


def kernel(node_feats, node_types, adj_indices, adj_values, idxes_seq,
           idxes_res, cosins, semantics, W0, b0, Wa_0, ba_0, gamma_0, beta_0,
           Wa_1, ba_1, gamma_1, beta_1, attn_W1, attn_b1, attn_W2, attn_b2):
    nt = node_types.reshape(N, 1)
    x0, x1 = _prologue(node_feats, nt, W0, b0.reshape(1, H),
                       Wa_0, ba_0.reshape(1, H), Wa_1, ba_1.reshape(1, H))
    xs = (x0, x1)
    s2s = []
    dep = None
    for m in range(2):
        xm = xs[m]
        xf = xm.reshape(2 * N, HH)
        k0 = idxes_seq[m, 0]
        k1 = idxes_seq[m, 1]
        r = idxes_res[m, 0]
        s1 = _spmm_sc(adj_indices[k0, 0], adj_indices[k0, 1],
                      adj_values[k0], xf, dep=dep)
        tbl = jnp.concatenate([s1.reshape(2, N, HH), xm],
                              axis=1).reshape(4 * N, HH)
        rows_c = jnp.concatenate([adj_indices[k1, 0], adj_indices[r, 0]])
        cols_c = jnp.concatenate([adj_indices[k1, 1], adj_indices[r, 1] + N])
        vals_c = jnp.concatenate([adj_values[k1], adj_values[r]])
        s2 = _spmm_sc(rows_c, cols_c, vals_c, tbl, dep=s1)
        s2s.append(s2.reshape(2, N, HH))
        dep = s2
    return _epilogue(s2s[0], s2s[1], gamma_0.reshape(1, H),
                     beta_0.reshape(1, H), gamma_1.reshape(1, H),
                     beta_1.reshape(1, H), attn_W1, attn_b1.reshape(1, H),
                     attn_W2, attn_b2.reshape(1, 1))


# speedup vs baseline: 1.7676x; 1.7380x over previous
import functools

import jax
import jax.numpy as jnp
from jax import lax
from jax.experimental import pallas as pl
from jax.experimental.pallas import tpu as pltpu
from jax.experimental.pallas import tpu_sc as plsc

N = 50000
DIN = 128
H = 64
HH = H // 2
NSUB = 16
NACC = 50048
ROWS_PER_TILE = NACC // NSUB
ZCH = 136
NZ = ROWS_PER_TILE // ZCH
C = 128
RBLK = 2000


def _spmm_body(ep, src, zeros, out, e_v, gath_v, acc,
               sem_st, sem_g, sem_sc):
    c = lax.axis_index("c")
    s = lax.axis_index("s")
    M = src.shape[0] // 2
    off = c * M
    CPT = ep.shape[0] // NSUB
    base = s * CPT

    row0 = pl.multiple_of(s * ROWS_PER_TILE, 8)
    pltpu.sync_copy(zeros, acc.at[pl.ds(row0, ROWS_PER_TILE)])
    plsc.subcore_barrier()

    def stage(k, eb):
        pltpu.async_copy(ep.at[base + k], e_v.at[eb], sem_st)

    def wait_stage():
        pltpu.make_async_copy(ep.at[base], e_v.at[0], sem_st).wait()

    def issue_gather(b, eb):
        @pl.when(c == 0)
        def _():
            pltpu.async_copy(src.at[e_v.at[eb, 1]], gath_v.at[b], sem_g)

        @pl.when(c == 1)
        def _():
            pltpu.async_copy(src.at[e_v.at[eb, 2]], gath_v.at[b], sem_g)

    def wait_gather():
        pltpu.make_async_copy(src.at[e_v.at[0, 1]], gath_v.at[0],
                              sem_g).wait()

    def issue_scatter(b, eb):
        pltpu.async_copy(gath_v.at[b], acc.at[e_v.at[eb, 0]], sem_sc,
                         add=True)

    def wait_scatter():
        pltpu.make_async_copy(gath_v.at[0], acc.at[e_v.at[0, 0]],
                              sem_sc).wait()

    stage(0, 0)
    stage(1, 1)
    wait_stage()
    issue_gather(0, 0)

    def iter_body(k, carry):
        b = lax.rem(k, 2)
        eb = lax.rem(k, 3)

        @pl.when(k + 1 < CPT)
        def _():
            wait_stage()

            @pl.when(k >= 1)
            def _():
                wait_scatter()

            issue_gather(lax.rem(k + 1, 2), lax.rem(k + 1, 3))

        @pl.when(k + 2 < CPT)
        def _():
            stage(k + 2, lax.rem(k + 2, 3))

        wait_gather()

        def cgrp(g, carry2):
            vv = plsc.bitcast(e_v[eb, 3, pl.ds(g * 16, 16)], jnp.float32)
            for t in range(16):
                v = vv[t]
                r = g * 16 + t
                for j in range(HH // 16):
                    sl = pl.ds(j * 16, 16)
                    gath_v[b, r, sl] = gath_v[b, r, sl] * v
            return carry2

        lax.fori_loop(0, C // 16, cgrp, 0)
        issue_scatter(b, eb)
        return carry

    lax.fori_loop(0, CPT, iter_body, 0)
    wait_scatter()
    wait_scatter()
    plsc.subcore_barrier()
    LAST = N - 15 * ROWS_PER_TILE

    @pl.when(s < NSUB - 1)
    def _():
        pltpu.sync_copy(acc.at[pl.ds(row0, ROWS_PER_TILE)],
                        out.at[pl.ds(c * N + row0, ROWS_PER_TILE)])

    @pl.when(s == NSUB - 1)
    def _():
        pltpu.sync_copy(acc.at[pl.ds(row0, LAST)],
                        out.at[pl.ds(c * N + row0, LAST)])


def _pack_edges(rows, cols, vals, M):
    E = rows.shape[0]
    Ep = -(-E // (NSUB * C)) * (NSUB * C)
    pad = Ep - E
    if pad:
        rows = jnp.concatenate([rows, jnp.zeros((pad,), jnp.int32)])
        cols = jnp.concatenate([cols, jnp.zeros((pad,), jnp.int32)])
        vals = jnp.concatenate([vals, jnp.zeros((pad,), jnp.float32)])
    vb = lax.bitcast_convert_type(vals, jnp.int32)
    return jnp.stack([rows.reshape(-1, C), cols.reshape(-1, C),
                      (cols + M).reshape(-1, C), vb.reshape(-1, C)], axis=1)


def _spmm_sc(rows, cols, vals, src, dep=None):
    M = src.shape[0] // 2
    ep = _pack_edges(rows, cols, vals, M)
    mesh = plsc.VectorSubcoreMesh(core_axis_name="c", subcore_axis_name="s")
    kern = functools.partial(
        pl.kernel,
        out_type=jax.ShapeDtypeStruct((2 * N, HH), jnp.float32),
        mesh=mesh,
        compiler_params=pltpu.CompilerParams(use_tc_tiling_on_sc=False,
                                             needs_layout_passes=False),
        scratch_types=[
            pltpu.VMEM((3, 4, C), jnp.int32),
            pltpu.VMEM((2, C, HH), jnp.float32),
            pltpu.VMEM_SHARED((NACC, HH), jnp.float32),
            pltpu.SemaphoreType.DMA,
            pltpu.SemaphoreType.DMA,
            pltpu.SemaphoreType.DMA,
        ],
    )(_spmm_body)
    zeros = jnp.zeros((ROWS_PER_TILE, HH), jnp.float32)
    if dep is not None:
        zeros, _ = lax.optimization_barrier((zeros, dep[0, :1]))
    return kern(ep, src, zeros)


def _prologue_body(nf, nt, W0, b0, Wa0, ba0, Wa1, ba1, x0_out, x1_out):
    hid = jnp.dot(nf[...], W0[...], preferred_element_type=jnp.float32)
    hid = hid + b0[...]
    hid = jnp.where(nt[...] == 0, hid, 0.0)
    x0 = jnp.dot(hid, Wa0[...], preferred_element_type=jnp.float32) + ba0[...]
    x1 = jnp.dot(hid, Wa1[...], preferred_element_type=jnp.float32) + ba1[...]
    x0_out[0] = x0[:, :HH]
    x0_out[1] = x0[:, HH:]
    x1_out[0] = x1[:, :HH]
    x1_out[1] = x1[:, HH:]


def _prologue(nf, nt, W0, b0, Wa0, ba0, Wa1, ba1):
    grid = (N // RBLK,)
    full = lambda shape: pl.BlockSpec(shape, lambda i: (0,) * len(shape))
    out_spec = pl.BlockSpec((2, RBLK, HH), lambda i: (0, i, 0))
    return pl.pallas_call(
        _prologue_body,
        grid=grid,
        in_specs=[
            pl.BlockSpec((RBLK, DIN), lambda i: (i, 0)),
            pl.BlockSpec((RBLK, 1), lambda i: (i, 0)),
            full((DIN, H)),
            full((1, H)),
            full((H, H)),
            full((1, H)),
            full((H, H)),
            full((1, H)),
        ],
        out_specs=[out_spec, out_spec],
        out_shape=[
            jax.ShapeDtypeStruct((2, N, HH), jnp.float32),
            jax.ShapeDtypeStruct((2, N, HH), jnp.float32),
        ],
    )(nf, nt, W0, b0, Wa0, ba0, Wa1, ba1)


def _layernorm(h, g, b):
    mu = jnp.mean(h, axis=-1, keepdims=True)
    var = jnp.mean((h - mu) ** 2, axis=-1, keepdims=True)
    return (h - mu) * lax.rsqrt(var + 1e-5) * g + b


def _epilogue_body(s20, s21, g0, bb0, g1, bb1, W1, b1, W2, b2, out):
    h0 = jnp.concatenate([s20[0], s20[1]], axis=-1)
    h1 = jnp.concatenate([s21[0], s21[1]], axis=-1)
    h0 = _layernorm(h0, g0[...], bb0[...])
    h1 = _layernorm(h1, g1[...], bb1[...])
    t0 = jnp.tanh(jnp.dot(h0, W1[...], preferred_element_type=jnp.float32)
                  + b1[...])
    t1 = jnp.tanh(jnp.dot(h1, W1[...], preferred_element_type=jnp.float32)
                  + b1[...])
    a0 = jnp.dot(t0, W2[...], preferred_element_type=jnp.float32) + b2[...]
    a1 = jnp.dot(t1, W2[...], preferred_element_type=jnp.float32) + b2[...]
    mx = jnp.maximum(a0, a1)
    e0 = jnp.exp(a0 - mx)
    e1 = jnp.exp(a1 - mx)
    z = e0 + e1
    out[...] = (e0 * h0 + e1 * h1) / z


def _epilogue(s20, s21, g0, bb0, g1, bb1, W1, b1, W2, b2):
    grid = (N // RBLK,)
    full = lambda shape: pl.BlockSpec(shape, lambda i: (0,) * len(shape))
    in_spec = pl.BlockSpec((2, RBLK, HH), lambda i: (0, i, 0))
    return pl.pallas_call(
        _epilogue_body,
        grid=grid,
        in_specs=[
            in_spec,
            in_spec,
            full((1, H)),
            full((1, H)),
            full((1, H)),
            full((1, H)),
            full((H, H)),
            full((1, H)),
            full((H, 1)),
            full((1, 1)),
        ],
        out_specs=pl.BlockSpec((RBLK, H), lambda i: (i, 0)),
        out_shape=jax.ShapeDtypeStruct((N, H), jnp.float32),
    )(s20, s21, g0, bb0, g1, bb1, W1, b1, W2, b2)


def kernel(node_feats, node_types, adj_indices, adj_values, idxes_seq,
           idxes_res, cosins, semantics, W0, b0, Wa_0, ba_0, gamma_0, beta_0,
           Wa_1, ba_1, gamma_1, beta_1, attn_W1, attn_b1, attn_W2, attn_b2):
    nt = node_types.reshape(N, 1)
    x0, x1 = _prologue(node_feats, nt, W0, b0.reshape(1, H),
                       Wa_0, ba_0.reshape(1, H), Wa_1, ba_1.reshape(1, H))
    xs = (x0, x1)
    s2s = []
    dep = None
    for m in range(2):
        xm = xs[m]
        xf = xm.reshape(2 * N, HH)
        k0 = idxes_seq[m, 0]
        k1 = idxes_seq[m, 1]
        r = idxes_res[m, 0]
        s1 = _spmm_sc(adj_indices[k0, 0], adj_indices[k0, 1],
                      adj_values[k0], xf, dep=dep)
        tbl = jnp.concatenate([s1.reshape(2, N, HH), xm],
                              axis=1).reshape(4 * N, HH)
        rows_c = jnp.concatenate([adj_indices[k1, 0], adj_indices[r, 0]])
        cols_c = jnp.concatenate([adj_indices[k1, 1], adj_indices[r, 1] + N])
        vals_c = jnp.concatenate([adj_values[k1], adj_values[r]])
        s2 = _spmm_sc(rows_c, cols_c, vals_c, tbl, dep=s1)
        s2s.append(s2.reshape(2, N, HH))
        dep = s2
    return _epilogue(s2s[0], s2s[1], gamma_0.reshape(1, H),
                     beta_0.reshape(1, H), gamma_1.reshape(1, H),
                     beta_1.reshape(1, H), attn_W1, attn_b1.reshape(1, H),
                     attn_W2, attn_b2.reshape(1, 1))


# speedup vs baseline: 1.7845x; 1.0096x over previous
import functools

import jax
import jax.numpy as jnp
from jax import lax
from jax.experimental import pallas as pl
from jax.experimental.pallas import tpu as pltpu
from jax.experimental.pallas import tpu_sc as plsc

N = 50000
DIN = 128
H = 64
HH = H // 2
NSUB = 16
NACC = 50048
ROWS_PER_TILE = NACC // NSUB
ZCH = 136
NZ = ROWS_PER_TILE // ZCH
C = 128
RBLK = 2000


def _spmm_body(ep, src, zeros, out, e_v, gath_v, acc,
               sem_st, sem_g, sem_sc):
    c = lax.axis_index("c")
    s = lax.axis_index("s")
    M = src.shape[0] // 2
    off = c * M
    CPT = ep.shape[0] // NSUB
    base = s * CPT

    row0 = pl.multiple_of(s * ROWS_PER_TILE, 8)
    pltpu.sync_copy(zeros, acc.at[pl.ds(row0, ROWS_PER_TILE)])
    plsc.subcore_barrier()

    def stage(k, eb):
        pltpu.async_copy(ep.at[base + k], e_v.at[eb], sem_st)

    def wait_stage():
        pltpu.make_async_copy(ep.at[base], e_v.at[0], sem_st).wait()

    def issue_gather(b, eb):
        @pl.when(c == 0)
        def _():
            pltpu.async_copy(src.at[e_v.at[eb, 1]], gath_v.at[b], sem_g)

        @pl.when(c == 1)
        def _():
            pltpu.async_copy(src.at[e_v.at[eb, 2]], gath_v.at[b], sem_g)

    def wait_gather():
        pltpu.make_async_copy(src.at[e_v.at[0, 1]], gath_v.at[0],
                              sem_g).wait()

    def issue_scatter(b, eb):
        pltpu.async_copy(gath_v.at[b], acc.at[e_v.at[eb, 0]], sem_sc,
                         add=True)

    def wait_scatter():
        pltpu.make_async_copy(gath_v.at[0], acc.at[e_v.at[0, 0]],
                              sem_sc).wait()

    stage(0, 0)
    stage(1, 1)
    wait_stage()
    issue_gather(0, 0)

    def iter_body(k, carry):
        b = lax.rem(k, 2)
        eb = lax.rem(k, 3)

        @pl.when(k + 1 < CPT)
        def _():
            wait_stage()

            @pl.when(k >= 1)
            def _():
                wait_scatter()

            issue_gather(lax.rem(k + 1, 2), lax.rem(k + 1, 3))

        @pl.when(k + 2 < CPT)
        def _():
            stage(k + 2, lax.rem(k + 2, 3))

        wait_gather()

        @plsc.parallel_loop(0, C // 16, unroll=2)
        def cgrp(g):
            vv = plsc.bitcast(e_v[eb, 3, pl.ds(g * 16, 16)], jnp.float32)
            for t in range(16):
                v = vv[t]
                r = g * 16 + t
                for j in range(HH // 16):
                    sl = pl.ds(j * 16, 16)
                    gath_v[b, r, sl] = gath_v[b, r, sl] * v
        issue_scatter(b, eb)
        return carry

    lax.fori_loop(0, CPT, iter_body, 0)
    wait_scatter()
    wait_scatter()
    plsc.subcore_barrier()
    LAST = N - 15 * ROWS_PER_TILE

    @pl.when(s < NSUB - 1)
    def _():
        pltpu.sync_copy(acc.at[pl.ds(row0, ROWS_PER_TILE)],
                        out.at[pl.ds(c * N + row0, ROWS_PER_TILE)])

    @pl.when(s == NSUB - 1)
    def _():
        pltpu.sync_copy(acc.at[pl.ds(row0, LAST)],
                        out.at[pl.ds(c * N + row0, LAST)])


def _pack_edges(rows, cols, vals, M):
    E = rows.shape[0]
    Ep = -(-E // (NSUB * C)) * (NSUB * C)
    pad = Ep - E
    if pad:
        rows = jnp.concatenate([rows, jnp.zeros((pad,), jnp.int32)])
        cols = jnp.concatenate([cols, jnp.zeros((pad,), jnp.int32)])
        vals = jnp.concatenate([vals, jnp.zeros((pad,), jnp.float32)])
    vb = lax.bitcast_convert_type(vals, jnp.int32)
    return jnp.stack([rows.reshape(-1, C), cols.reshape(-1, C),
                      (cols + M).reshape(-1, C), vb.reshape(-1, C)], axis=1)


def _spmm_sc(rows, cols, vals, src, dep=None):
    M = src.shape[0] // 2
    ep = _pack_edges(rows, cols, vals, M)
    mesh = plsc.VectorSubcoreMesh(core_axis_name="c", subcore_axis_name="s")
    kern = functools.partial(
        pl.kernel,
        out_type=jax.ShapeDtypeStruct((2 * N, HH), jnp.float32),
        mesh=mesh,
        compiler_params=pltpu.CompilerParams(use_tc_tiling_on_sc=False,
                                             needs_layout_passes=False),
        scratch_types=[
            pltpu.VMEM((3, 4, C), jnp.int32),
            pltpu.VMEM((2, C, HH), jnp.float32),
            pltpu.VMEM_SHARED((NACC, HH), jnp.float32),
            pltpu.SemaphoreType.DMA,
            pltpu.SemaphoreType.DMA,
            pltpu.SemaphoreType.DMA,
        ],
    )(_spmm_body)
    zeros = jnp.zeros((ROWS_PER_TILE, HH), jnp.float32)
    if dep is not None:
        zeros, _ = lax.optimization_barrier((zeros, dep[0, :1]))
    return kern(ep, src, zeros)


def _prologue_body(nf, nt, W0, b0, Wa0, ba0, Wa1, ba1, x0_out, x1_out):
    hid = jnp.dot(nf[...], W0[...], preferred_element_type=jnp.float32)
    hid = hid + b0[...]
    hid = jnp.where(nt[...] == 0, hid, 0.0)
    x0 = jnp.dot(hid, Wa0[...], preferred_element_type=jnp.float32) + ba0[...]
    x1 = jnp.dot(hid, Wa1[...], preferred_element_type=jnp.float32) + ba1[...]
    x0_out[0] = x0[:, :HH]
    x0_out[1] = x0[:, HH:]
    x1_out[0] = x1[:, :HH]
    x1_out[1] = x1[:, HH:]


def _prologue(nf, nt, W0, b0, Wa0, ba0, Wa1, ba1):
    grid = (N // RBLK,)
    full = lambda shape: pl.BlockSpec(shape, lambda i: (0,) * len(shape))
    out_spec = pl.BlockSpec((2, RBLK, HH), lambda i: (0, i, 0))
    return pl.pallas_call(
        _prologue_body,
        grid=grid,
        in_specs=[
            pl.BlockSpec((RBLK, DIN), lambda i: (i, 0)),
            pl.BlockSpec((RBLK, 1), lambda i: (i, 0)),
            full((DIN, H)),
            full((1, H)),
            full((H, H)),
            full((1, H)),
            full((H, H)),
            full((1, H)),
        ],
        out_specs=[out_spec, out_spec],
        out_shape=[
            jax.ShapeDtypeStruct((2, N, HH), jnp.float32),
            jax.ShapeDtypeStruct((2, N, HH), jnp.float32),
        ],
    )(nf, nt, W0, b0, Wa0, ba0, Wa1, ba1)


def _layernorm(h, g, b):
    mu = jnp.mean(h, axis=-1, keepdims=True)
    var = jnp.mean((h - mu) ** 2, axis=-1, keepdims=True)
    return (h - mu) * lax.rsqrt(var + 1e-5) * g + b


def _epilogue_body(s20, s21, g0, bb0, g1, bb1, W1, b1, W2, b2, out):
    h0 = jnp.concatenate([s20[0], s20[1]], axis=-1)
    h1 = jnp.concatenate([s21[0], s21[1]], axis=-1)
    h0 = _layernorm(h0, g0[...], bb0[...])
    h1 = _layernorm(h1, g1[...], bb1[...])
    t0 = jnp.tanh(jnp.dot(h0, W1[...], preferred_element_type=jnp.float32)
                  + b1[...])
    t1 = jnp.tanh(jnp.dot(h1, W1[...], preferred_element_type=jnp.float32)
                  + b1[...])
    a0 = jnp.dot(t0, W2[...], preferred_element_type=jnp.float32) + b2[...]
    a1 = jnp.dot(t1, W2[...], preferred_element_type=jnp.float32) + b2[...]
    mx = jnp.maximum(a0, a1)
    e0 = jnp.exp(a0 - mx)
    e1 = jnp.exp(a1 - mx)
    z = e0 + e1
    out[...] = (e0 * h0 + e1 * h1) / z


def _epilogue(s20, s21, g0, bb0, g1, bb1, W1, b1, W2, b2):
    grid = (N // RBLK,)
    full = lambda shape: pl.BlockSpec(shape, lambda i: (0,) * len(shape))
    in_spec = pl.BlockSpec((2, RBLK, HH), lambda i: (0, i, 0))
    return pl.pallas_call(
        _epilogue_body,
        grid=grid,
        in_specs=[
            in_spec,
            in_spec,
            full((1, H)),
            full((1, H)),
            full((1, H)),
            full((1, H)),
            full((H, H)),
            full((1, H)),
            full((H, 1)),
            full((1, 1)),
        ],
        out_specs=pl.BlockSpec((RBLK, H), lambda i: (i, 0)),
        out_shape=jax.ShapeDtypeStruct((N, H), jnp.float32),
    )(s20, s21, g0, bb0, g1, bb1, W1, b1, W2, b2)


def kernel(node_feats, node_types, adj_indices, adj_values, idxes_seq,
           idxes_res, cosins, semantics, W0, b0, Wa_0, ba_0, gamma_0, beta_0,
           Wa_1, ba_1, gamma_1, beta_1, attn_W1, attn_b1, attn_W2, attn_b2):
    nt = node_types.reshape(N, 1)
    x0, x1 = _prologue(node_feats, nt, W0, b0.reshape(1, H),
                       Wa_0, ba_0.reshape(1, H), Wa_1, ba_1.reshape(1, H))
    xs = (x0, x1)
    s2s = []
    dep = None
    for m in range(2):
        xm = xs[m]
        xf = xm.reshape(2 * N, HH)
        k0 = idxes_seq[m, 0]
        k1 = idxes_seq[m, 1]
        r = idxes_res[m, 0]
        s1 = _spmm_sc(adj_indices[k0, 0], adj_indices[k0, 1],
                      adj_values[k0], xf, dep=dep)
        tbl = jnp.concatenate([s1.reshape(2, N, HH), xm],
                              axis=1).reshape(4 * N, HH)
        rows_c = jnp.concatenate([adj_indices[k1, 0], adj_indices[r, 0]])
        cols_c = jnp.concatenate([adj_indices[k1, 1], adj_indices[r, 1] + N])
        vals_c = jnp.concatenate([adj_values[k1], adj_values[r]])
        s2 = _spmm_sc(rows_c, cols_c, vals_c, tbl, dep=s1)
        s2s.append(s2.reshape(2, N, HH))
        dep = s2
    return _epilogue(s2s[0], s2s[1], gamma_0.reshape(1, H),
                     beta_0.reshape(1, H), gamma_1.reshape(1, H),
                     beta_1.reshape(1, H), attn_W1, attn_b1.reshape(1, H),
                     attn_W2, attn_b2.reshape(1, 1))
